# baseline (speedup 1.0000x reference)
import functools

import jax
import jax.numpy as jnp
import numpy as np
from jax import lax
from jax.experimental import pallas as pl
from jax.experimental.pallas import tpu as pltpu
from jax.experimental.pallas import tpu_sc as plsc

N = 10000
E = 320000
D = 128
NC = 2
NS = 16
NW = NC * NS
K = 40
CW = E // (K * NW)
GC = 10
GPW = CW // GC
OROWS = 624
ZROWS = 40
LANES = 16
DG = D // LANES

_PERM = np.empty((D,), dtype=np.int32)
for _j in range(D // 32):
    for _t in range(16):
        _PERM[32 * _j + 2 * _t] = 32 * _j + _t
        _PERM[32 * _j + 2 * _t + 1] = 32 * _j + 16 + _t


def _sc_aggregate(xpk, idx3d, attr):
    mesh = plsc.VectorSubcoreMesh(core_axis_name="c", subcore_axis_name="s")

    @functools.partial(
        pl.kernel,
        out_type=jax.ShapeDtypeStruct((NC, N, D), jnp.float32),
        mesh=mesh,
        compiler_params=pltpu.CompilerParams(use_tc_tiling_on_sc=False),
        scratch_types=[
            pltpu.VMEM((2, 2 * GC, K), jnp.int32),
            pltpu.VMEM((K, D), jnp.float32),
            pltpu.VMEM((K, D), jnp.float32),
            pltpu.VMEM((K, D // 2), jnp.int32),
            pltpu.VMEM((K, D // 2), jnp.int32),
            pltpu.VMEM((K, D), jnp.float32),
            pltpu.VMEM((K, D), jnp.float32),
            pltpu.VMEM_SHARED((N, D), jnp.float32),
            pltpu.SemaphoreType.DMA,
            pltpu.SemaphoreType.DMA,
            pltpu.SemaphoreType.DMA,
            pltpu.SemaphoreType.DMA,
            pltpu.SemaphoreType.DMA,
            pltpu.SemaphoreType.DMA,
            pltpu.SemaphoreType.DMA,
        ],
    )
    def agg_kernel(x_hbm, idx_hbm, attr_hbm, out_hbm,
                   idxb, ab0, ab1, xb0, xb1, mb0, mb1, accum,
                   si, sg0, sg1, sa0, sa1, ss0, ss1):
        c = lax.axis_index("c")
        s = lax.axis_index("s")
        wid = s * NC + c

        pltpu.sync_copy(idx_hbm.at[wid * GPW], idxb.at[0])

        def early_gather(ci, xb, ab, sg, sa):
            pltpu.async_copy(x_hbm.at[idxb.at[0, ci]], xb, sg)
            pltpu.async_copy(attr_hbm.at[pl.ds((wid * CW + ci) * K, K)], ab,
                             sa)

        early_gather(0, xb0, ab0, sg0, sa0)
        early_gather(1, xb1, ab1, sg1, sa1)

        zv = jnp.zeros((LANES,), jnp.float32)

        def zrow(i, carry):
            for j in range(DG):
                mb0[i, pl.ds(j * LANES, LANES)] = zv
            return carry

        lax.fori_loop(0, ZROWS, zrow, 0)

        def zcp(r, carry):
            pltpu.sync_copy(mb0, accum.at[pl.ds(s * OROWS + r * ZROWS, ZROWS)])
            return carry

        lax.fori_loop(0, 16, zcp, 0)

        plsc.subcore_barrier()

        def idx_start(g):
            pltpu.async_copy(idx_hbm.at[wid * GPW + g], idxb.at[g % 2], si)

        def idx_wait(g):
            pltpu.make_async_copy(idx_hbm.at[wid * GPW + g],
                                  idxb.at[g % 2], si).wait()

        def src_row(ci):
            return idxb.at[(ci // GC) % 2, ci % GC]

        def dst_row(ci):
            return idxb.at[(ci // GC) % 2, GC + ci % GC]

        def gather_start(ci, xb, ab, sg, sa):
            pltpu.async_copy(x_hbm.at[src_row(ci)], xb, sg)
            pltpu.async_copy(attr_hbm.at[pl.ds((wid * CW + ci) * K, K)], ab, sa)

        def gather_wait(ci, xb, ab, sg, sa):
            pltpu.make_async_copy(x_hbm.at[src_row(ci)], xb, sg).wait()
            pltpu.make_async_copy(
                attr_hbm.at[pl.ds((wid * CW + ci) * K, K)], ab, sa).wait()

        def scatter_start(ci, mb, ss):
            pltpu.async_copy(mb, accum.at[dst_row(ci)], ss, add=True)

        def scatter_wait(ci, mb, ss):
            pltpu.make_async_copy(mb, accum.at[dst_row(ci)], ss).wait()

        def compute(xb, ab, mb):
            himask = jnp.full((LANES,), jnp.int32(-65536))

            def body(e2, carry2):
                for ee in range(2):
                    e = 2 * e2 + ee
                    for j in range(DG // 2):
                        v = xb[e, pl.ds(j * LANES, LANES)]
                        a = lax.bitcast_convert_type(v << 16, jnp.float32)
                        b = lax.bitcast_convert_type(v & himask, jnp.float32)
                        sla = pl.ds(j * 2 * LANES, LANES)
                        slb = pl.ds(j * 2 * LANES + LANES, LANES)
                        mb[e, sla] = jnp.maximum(a + ab[e, sla], 0.0)
                        mb[e, slb] = jnp.maximum(b + ab[e, slb], 0.0)
                return carry2

            lax.fori_loop(0, K // 2, body, 0)

        def half(i, c0, ab, xb, mb, sg, sa, ss):
            gather_wait(c0, xb, ab, sg, sa)

            @pl.when(i > 0)
            def _():
                scatter_wait(c0, mb, ss)

            compute(xb, ab, mb)
            scatter_start(c0, mb, ss)

            @pl.when(i < CW // 2 - 1)
            def _():
                gather_start(c0 + 2, xb, ab, sg, sa)

        def outer(i, carry):
            c0 = 2 * i

            @pl.when(jnp.logical_and(i % (GC // 2) == GC // 2 - 1,
                                     i < CW // 2 - 1))
            def _():
                idx_wait(i // (GC // 2) + 1)

            @pl.when(jnp.logical_and(i % (GC // 2) == 1,
                                     i < (GPW - 1) * (GC // 2)))
            def _():
                idx_start(i // (GC // 2) + 1)

            half(i, c0, ab0, xb0, mb0, sg0, sa0, ss0)
            half(i, c0 + 1, ab1, xb1, mb1, sg1, sa1, ss1)
            return carry

        lax.fori_loop(0, CW // 2, outer, 0)

        scatter_wait(CW - 2, mb0, ss0)
        scatter_wait(CW - 1, mb1, ss1)

        plsc.subcore_barrier()

        pltpu.sync_copy(accum.at[pl.ds(s * OROWS, OROWS)],
                        out_hbm.at[c, pl.ds(s * OROWS, OROWS)])

        @pl.when(s == 0)
        def _():
            pltpu.sync_copy(accum.at[pl.ds(NS * OROWS, N - NS * OROWS)],
                            out_hbm.at[c, pl.ds(NS * OROWS, N - NS * OROWS)])

    return agg_kernel(xpk, idx3d, attr)


def _tc_mlp(x, parts, W1, b1, gamma, beta, W2, b2):
    def body(x_ref, p_ref, w1_ref, b1_ref, g_ref, be_ref, w2_ref,
             b2_ref, o_ref):
        xv = x_ref[...]
        h0 = xv + p_ref[0] + p_ref[1]
        h = lax.dot_general(h0, w1_ref[...], (((1,), (1,)), ((), ())),
                            preferred_element_type=jnp.float32) + b1_ref[...]
        mean = jnp.mean(h, axis=0, keepdims=True)
        cent = h - mean
        var = jnp.mean(cent * cent, axis=0, keepdims=True)
        hn = cent * lax.rsqrt(var + 1e-5) * g_ref[...] + be_ref[...]
        hr = jnp.maximum(hn, 0.0)
        h2 = lax.dot_general(hr, w2_ref[...], (((1,), (1,)), ((), ())),
                             preferred_element_type=jnp.float32) + b2_ref[...]
        o_ref[...] = xv + jnp.maximum(h2, 0.0)

    return pl.pallas_call(
        body,
        out_shape=jax.ShapeDtypeStruct((N, D), jnp.float32),
    )(x, parts, W1, b1, gamma, beta, W2, b2)


def kernel(x, edge_index, edge_attr, W1, b1, gamma, beta, W2, b2):
    src3d = edge_index[0].astype(jnp.int32).reshape(NW * GPW, GC, K)
    dst3d = edge_index[1].astype(jnp.int32).reshape(NW * GPW, GC, K)
    idx3d = jnp.concatenate([src3d, dst3d], axis=1)
    xb16 = x.astype(jnp.bfloat16)[:, _PERM]
    xpk = jax.lax.bitcast_convert_type(
        xb16.reshape(N, D // 2, 2), jnp.int32)
    parts = _sc_aggregate(xpk, idx3d, edge_attr)
    return _tc_mlp(x, parts, W1,
                   b1.reshape(1, D), gamma.reshape(1, D), beta.reshape(1, D),
                   W2, b2.reshape(1, D))


# speedup vs baseline: 1.4673x; 1.4673x over previous
import functools

import jax
import jax.numpy as jnp
from jax import lax
from jax.experimental import pallas as pl
from jax.experimental.pallas import tpu as pltpu
from jax.experimental.pallas import tpu_sc as plsc

N = 10000
E = 320000
D = 128
NC = 2
NS = 16
NW = NC * NS
K = 40
CW = E // (K * NW)
GC = 10
GPW = CW // GC
OROWS = 624
ZROWS = 40
LANES = 16
DG = D // LANES


def _sc_aggregate(x, idx3d, attr):
    mesh = plsc.VectorSubcoreMesh(core_axis_name="c", subcore_axis_name="s")

    @functools.partial(
        pl.kernel,
        out_type=jax.ShapeDtypeStruct((NC, N, D), jnp.float32),
        mesh=mesh,
        scratch_types=[
            pltpu.VMEM((2, 2 * GC, K), jnp.int32),
            pltpu.VMEM((K, D), jnp.float32),
            pltpu.VMEM((K, D), jnp.float32),
            pltpu.VMEM((K, D), jnp.float32),
            pltpu.VMEM((K, D), jnp.float32),
            pltpu.VMEM((K, D), jnp.float32),
            pltpu.VMEM((K, D), jnp.float32),
            pltpu.VMEM_SHARED((N, D), jnp.float32),
            pltpu.SemaphoreType.DMA,
            pltpu.SemaphoreType.DMA,
            pltpu.SemaphoreType.DMA,
            pltpu.SemaphoreType.DMA,
            pltpu.SemaphoreType.DMA,
            pltpu.SemaphoreType.DMA,
            pltpu.SemaphoreType.DMA,
        ],
    )
    def agg_kernel(x_hbm, idx_hbm, attr_hbm, out_hbm,
                   idxb, ab0, ab1, xb0, xb1, mb0, mb1, accum,
                   si, sg0, sg1, sa0, sa1, ss0, ss1):
        c = lax.axis_index("c")
        s = lax.axis_index("s")
        wid = s * NC + c

        pltpu.sync_copy(idx_hbm.at[wid * GPW], idxb.at[0])

        def early_gather(ci, xb, ab, sg, sa):
            pltpu.async_copy(x_hbm.at[idxb.at[0, ci]], xb, sg)
            pltpu.async_copy(attr_hbm.at[pl.ds((wid * CW + ci) * K, K)], ab,
                             sa)

        early_gather(0, xb0, ab0, sg0, sa0)
        early_gather(1, xb1, ab1, sg1, sa1)

        zv = jnp.zeros((LANES,), jnp.float32)

        def zrow(i, carry):
            for j in range(DG):
                mb0[i, pl.ds(j * LANES, LANES)] = zv
            return carry

        lax.fori_loop(0, ZROWS, zrow, 0)

        def zcp(r, carry):
            pltpu.sync_copy(mb0, accum.at[pl.ds(s * OROWS + r * ZROWS, ZROWS)])
            return carry

        lax.fori_loop(0, 16, zcp, 0)

        plsc.subcore_barrier()

        def idx_start(g):
            pltpu.async_copy(idx_hbm.at[wid * GPW + g], idxb.at[g % 2], si)

        def idx_wait(g):
            pltpu.make_async_copy(idx_hbm.at[wid * GPW + g],
                                  idxb.at[g % 2], si).wait()

        def src_row(ci):
            return idxb.at[(ci // GC) % 2, ci % GC]

        def dst_row(ci):
            return idxb.at[(ci // GC) % 2, GC + ci % GC]

        def gather_start(ci, xb, ab, sg, sa):
            pltpu.async_copy(x_hbm.at[src_row(ci)], xb, sg)
            pltpu.async_copy(attr_hbm.at[pl.ds((wid * CW + ci) * K, K)], ab, sa)

        def gather_wait(ci, xb, ab, sg, sa):
            pltpu.make_async_copy(x_hbm.at[src_row(ci)], xb, sg).wait()
            pltpu.make_async_copy(
                attr_hbm.at[pl.ds((wid * CW + ci) * K, K)], ab, sa).wait()

        def scatter_start(ci, mb, ss):
            pltpu.async_copy(mb, accum.at[dst_row(ci)], ss, add=True)

        def scatter_wait(ci, mb, ss):
            pltpu.make_async_copy(mb, accum.at[dst_row(ci)], ss).wait()

        def compute(xb, ab, mb):
            def body(e2, carry2):
                for ee in range(2):
                    e = 2 * e2 + ee
                    for j in range(DG):
                        sl = pl.ds(j * LANES, LANES)
                        mb[e, sl] = jnp.maximum(xb[e, sl] + ab[e, sl], 0.0)
                return carry2

            lax.fori_loop(0, K // 2, body, 0)

        def half(i, c0, ab, xb, mb, sg, sa, ss):
            gather_wait(c0, xb, ab, sg, sa)

            @pl.when(i > 0)
            def _():
                scatter_wait(c0, mb, ss)

            compute(xb, ab, mb)
            scatter_start(c0, mb, ss)

            @pl.when(i < CW // 2 - 1)
            def _():
                gather_start(c0 + 2, xb, ab, sg, sa)

        def outer(i, carry):
            c0 = 2 * i

            @pl.when(jnp.logical_and(i % (GC // 2) == GC // 2 - 1,
                                     i < CW // 2 - 1))
            def _():
                idx_wait(i // (GC // 2) + 1)

            @pl.when(jnp.logical_and(i % (GC // 2) == 1,
                                     i < (GPW - 1) * (GC // 2)))
            def _():
                idx_start(i // (GC // 2) + 1)

            half(i, c0, ab0, xb0, mb0, sg0, sa0, ss0)
            half(i, c0 + 1, ab1, xb1, mb1, sg1, sa1, ss1)
            return carry

        lax.fori_loop(0, CW // 2, outer, 0)

        scatter_wait(CW - 2, mb0, ss0)
        scatter_wait(CW - 1, mb1, ss1)

        plsc.subcore_barrier()

        pltpu.sync_copy(accum.at[pl.ds(s * OROWS, OROWS)],
                        out_hbm.at[c, pl.ds(s * OROWS, OROWS)])

        @pl.when(s == 0)
        def _():
            pltpu.sync_copy(accum.at[pl.ds(NS * OROWS, N - NS * OROWS)],
                            out_hbm.at[c, pl.ds(NS * OROWS, N - NS * OROWS)])

    return agg_kernel(x, idx3d, attr)


def _tc_mlp(x, parts, W1, b1, gamma, beta, W2, b2):
    def body(x_ref, p_ref, w1_ref, b1_ref, g_ref, be_ref, w2_ref,
             b2_ref, o_ref):
        xv = x_ref[...]
        h0 = xv + p_ref[0] + p_ref[1]
        h = lax.dot_general(h0, w1_ref[...], (((1,), (1,)), ((), ())),
                            preferred_element_type=jnp.float32) + b1_ref[...]
        mean = jnp.mean(h, axis=0, keepdims=True)
        cent = h - mean
        var = jnp.mean(cent * cent, axis=0, keepdims=True)
        hn = cent * lax.rsqrt(var + 1e-5) * g_ref[...] + be_ref[...]
        hr = jnp.maximum(hn, 0.0)
        h2 = lax.dot_general(hr, w2_ref[...], (((1,), (1,)), ((), ())),
                             preferred_element_type=jnp.float32) + b2_ref[...]
        o_ref[...] = xv + jnp.maximum(h2, 0.0)

    return pl.pallas_call(
        body,
        out_shape=jax.ShapeDtypeStruct((N, D), jnp.float32),
    )(x, parts, W1, b1, gamma, beta, W2, b2)


def kernel(x, edge_index, edge_attr, W1, b1, gamma, beta, W2, b2):
    src3d = edge_index[0].astype(jnp.int32).reshape(NW * GPW, GC, K)
    dst3d = edge_index[1].astype(jnp.int32).reshape(NW * GPW, GC, K)
    idx3d = jnp.concatenate([src3d, dst3d], axis=1)
    parts = _sc_aggregate(x, idx3d, edge_attr)
    return _tc_mlp(x, parts, W1,
                   b1.reshape(1, D), gamma.reshape(1, D), beta.reshape(1, D),
                   W2, b2.reshape(1, D))


# speedup vs baseline: 1.4730x; 1.0039x over previous
import functools

import jax
import jax.numpy as jnp
from jax import lax
from jax.experimental import pallas as pl
from jax.experimental.pallas import tpu as pltpu
from jax.experimental.pallas import tpu_sc as plsc

N = 10000
E = 320000
D = 128
NC = 2
NS = 16
NW = NC * NS
K = 40
CW = E // (K * NW)
GC = 10
GPW = CW // GC
OROWS = 624
ZROWS = 40
LANES = 16
DG = D // LANES


def _sc_aggregate(x, idx3d, attr):
    mesh = plsc.VectorSubcoreMesh(core_axis_name="c", subcore_axis_name="s")

    @functools.partial(
        pl.kernel,
        out_type=jax.ShapeDtypeStruct((NC, N, D), jnp.float32),
        mesh=mesh,
        scratch_types=[
            pltpu.VMEM((2, 2 * GC, K), jnp.int32),
            pltpu.VMEM((K, D), jnp.float32),
            pltpu.VMEM((K, D), jnp.float32),
            pltpu.VMEM((K, D), jnp.float32),
            pltpu.VMEM((K, D), jnp.float32),
            pltpu.VMEM((K, D), jnp.float32),
            pltpu.VMEM((K, D), jnp.float32),
            pltpu.VMEM_SHARED((N, D), jnp.float32),
            pltpu.SemaphoreType.DMA,
            pltpu.SemaphoreType.DMA,
            pltpu.SemaphoreType.DMA,
            pltpu.SemaphoreType.DMA,
            pltpu.SemaphoreType.DMA,
            pltpu.SemaphoreType.DMA,
            pltpu.SemaphoreType.DMA,
        ],
    )
    def agg_kernel(x_hbm, idx_hbm, attr_hbm, out_hbm,
                   idxb, ab0, ab1, xb0, xb1, mb0, mb1, accum,
                   si, sg0, sg1, sa0, sa1, ss0, ss1):
        c = lax.axis_index("c")
        s = lax.axis_index("s")
        wid = s * NC + c

        pltpu.sync_copy(idx_hbm.at[wid * GPW], idxb.at[0])

        def early_gather(ci, xb, ab, sg, sa):
            pltpu.async_copy(x_hbm.at[idxb.at[0, ci]], xb, sg)
            pltpu.async_copy(attr_hbm.at[pl.ds((wid * CW + ci) * K, K)], ab,
                             sa)

        early_gather(0, xb0, ab0, sg0, sa0)
        early_gather(1, xb1, ab1, sg1, sa1)

        zv = jnp.zeros((LANES,), jnp.float32)

        def zrow(i, carry):
            for j in range(DG):
                mb0[i, pl.ds(j * LANES, LANES)] = zv
            return carry

        lax.fori_loop(0, ZROWS, zrow, 0)

        def zcp(r, carry):
            pltpu.async_copy(
                mb0, accum.at[pl.ds(s * OROWS + r * ZROWS, ZROWS)], si)
            return carry

        def zcp_wait(r, carry):
            pltpu.make_async_copy(
                mb0, accum.at[pl.ds(s * OROWS + r * ZROWS, ZROWS)], si).wait()
            return carry

        lax.fori_loop(0, 16, zcp, 0)
        lax.fori_loop(0, 16, zcp_wait, 0)

        plsc.subcore_barrier()

        def idx_start(g):
            pltpu.async_copy(idx_hbm.at[wid * GPW + g], idxb.at[g % 2], si)

        def idx_wait(g):
            pltpu.make_async_copy(idx_hbm.at[wid * GPW + g],
                                  idxb.at[g % 2], si).wait()

        def src_row(ci):
            return idxb.at[(ci // GC) % 2, ci % GC]

        def dst_row(ci):
            return idxb.at[(ci // GC) % 2, GC + ci % GC]

        def gather_start(ci, xb, ab, sg, sa):
            pltpu.async_copy(x_hbm.at[src_row(ci)], xb, sg)
            pltpu.async_copy(attr_hbm.at[pl.ds((wid * CW + ci) * K, K)], ab, sa)

        def gather_wait(ci, xb, ab, sg, sa):
            pltpu.make_async_copy(x_hbm.at[src_row(ci)], xb, sg).wait()
            pltpu.make_async_copy(
                attr_hbm.at[pl.ds((wid * CW + ci) * K, K)], ab, sa).wait()

        def scatter_start(ci, mb, ss):
            pltpu.async_copy(mb, accum.at[dst_row(ci)], ss, add=True)

        def scatter_wait(ci, mb, ss):
            pltpu.make_async_copy(mb, accum.at[dst_row(ci)], ss).wait()

        def compute(xb, ab, mb):
            def body(e4, carry2):
                for ee in range(4):
                    e = 4 * e4 + ee
                    for j in range(DG):
                        sl = pl.ds(j * LANES, LANES)
                        mb[e, sl] = jnp.maximum(xb[e, sl] + ab[e, sl], 0.0)
                return carry2

            lax.fori_loop(0, K // 4, body, 0)

        def half(i, c0, ab, xb, mb, sg, sa, ss):
            gather_wait(c0, xb, ab, sg, sa)

            @pl.when(i > 0)
            def _():
                scatter_wait(c0, mb, ss)

            compute(xb, ab, mb)
            scatter_start(c0, mb, ss)

            @pl.when(i < CW // 2 - 1)
            def _():
                gather_start(c0 + 2, xb, ab, sg, sa)

        def outer(i, carry):
            c0 = 2 * i

            @pl.when(jnp.logical_and(i % (GC // 2) == GC // 2 - 1,
                                     i < CW // 2 - 1))
            def _():
                idx_wait(i // (GC // 2) + 1)

            @pl.when(jnp.logical_and(i % (GC // 2) == 1,
                                     i < (GPW - 1) * (GC // 2)))
            def _():
                idx_start(i // (GC // 2) + 1)

            half(i, c0, ab0, xb0, mb0, sg0, sa0, ss0)
            half(i, c0 + 1, ab1, xb1, mb1, sg1, sa1, ss1)
            return carry

        lax.fori_loop(0, CW // 2, outer, 0)

        scatter_wait(CW - 2, mb0, ss0)
        scatter_wait(CW - 1, mb1, ss1)

        plsc.subcore_barrier()

        pltpu.sync_copy(accum.at[pl.ds(s * OROWS, OROWS)],
                        out_hbm.at[c, pl.ds(s * OROWS, OROWS)])

        @pl.when(s == 0)
        def _():
            pltpu.sync_copy(accum.at[pl.ds(NS * OROWS, N - NS * OROWS)],
                            out_hbm.at[c, pl.ds(NS * OROWS, N - NS * OROWS)])

    return agg_kernel(x, idx3d, attr)


def _tc_mlp(x, parts, W1, b1, gamma, beta, W2, b2):
    def body(x_ref, p_ref, w1_ref, b1_ref, g_ref, be_ref, w2_ref,
             b2_ref, o_ref):
        xv = x_ref[...]
        h0 = xv + p_ref[0] + p_ref[1]
        h = lax.dot_general(h0, w1_ref[...], (((1,), (1,)), ((), ())),
                            preferred_element_type=jnp.float32) + b1_ref[...]
        mean = jnp.mean(h, axis=0, keepdims=True)
        cent = h - mean
        var = jnp.mean(cent * cent, axis=0, keepdims=True)
        hn = cent * lax.rsqrt(var + 1e-5) * g_ref[...] + be_ref[...]
        hr = jnp.maximum(hn, 0.0)
        h2 = lax.dot_general(hr, w2_ref[...], (((1,), (1,)), ((), ())),
                             preferred_element_type=jnp.float32) + b2_ref[...]
        o_ref[...] = xv + jnp.maximum(h2, 0.0)

    return pl.pallas_call(
        body,
        out_shape=jax.ShapeDtypeStruct((N, D), jnp.float32),
    )(x, parts, W1, b1, gamma, beta, W2, b2)


def kernel(x, edge_index, edge_attr, W1, b1, gamma, beta, W2, b2):
    src3d = edge_index[0].astype(jnp.int32).reshape(NW * GPW, GC, K)
    dst3d = edge_index[1].astype(jnp.int32).reshape(NW * GPW, GC, K)
    idx3d = jnp.concatenate([src3d, dst3d], axis=1)
    parts = _sc_aggregate(x, idx3d, edge_attr)
    return _tc_mlp(x, parts, W1,
                   b1.reshape(1, D), gamma.reshape(1, D), beta.reshape(1, D),
                   W2, b2.reshape(1, D))
